# baseline (speedup 1.0000x reference)
import functools

import jax
import jax.numpy as jnp
from jax import lax
from jax.experimental import pallas as pl
from jax.experimental.pallas import tpu as pltpu
from jax.experimental.pallas import tpu_sc as plsc

_NUM_CORES = 2
_NUM_SUBCORES = 16
_NUM_WORKERS = _NUM_CORES * _NUM_SUBCORES


@functools.lru_cache(maxsize=None)
def _make_gather(V, D, B, L, nbuf=4):
    b_per_w = B // _NUM_WORKERS
    n_groups = b_per_w // nbuf
    assert B % _NUM_WORKERS == 0 and b_per_w % nbuf == 0

    mesh = plsc.VectorSubcoreMesh(
        core_axis_name="c", subcore_axis_name="s",
        num_cores=_NUM_CORES, num_subcores=_NUM_SUBCORES)

    @functools.partial(
        pl.kernel,
        out_type=jax.ShapeDtypeStruct((B, L, D), jnp.float32),
        mesh=mesh,
        compiler_params=pltpu.CompilerParams(use_tc_tiling_on_sc=False),
        scratch_types=[
            pltpu.VMEM((b_per_w, L), jnp.int32),
            [pltpu.VMEM((L, D), jnp.float32) for _ in range(nbuf)],
            [pltpu.SemaphoreType.DMA for _ in range(nbuf)],
            [pltpu.SemaphoreType.DMA for _ in range(nbuf)],
        ],
    )
    def gather_kernel(idx_hbm, table_hbm, out_hbm, idx_v, rows, gsem, wsem):
        wid = lax.axis_index("s") * _NUM_CORES + lax.axis_index("c")
        base = wid * b_per_w
        pltpu.sync_copy(idx_hbm.at[pl.ds(base, b_per_w)], idx_v)

        def fire_gather(c, b):
            pltpu.async_copy(table_hbm.at[idx_v.at[c]], rows[b], gsem[b])

        def wait_gather(c, b):
            pltpu.make_async_copy(
                table_hbm.at[idx_v.at[c]], rows[b], gsem[b]).wait()

        def fire_write(c, b):
            pltpu.async_copy(rows[b], out_hbm.at[base + c], wsem[b])

        def wait_write(c, b):
            pltpu.make_async_copy(
                rows[b], out_hbm.at[base + c], wsem[b]).wait()

        for b in range(nbuf):
            fire_gather(b, b)

        def body(i, carry):
            cc = i * nbuf
            for b in range(nbuf):
                c = cc + b
                wait_gather(c, b)
                fire_write(c, b)
                wait_write(c, b)
                fire_gather(c + nbuf, b)
            return carry

        lax.fori_loop(0, n_groups - 1, body, 0)

        last = (n_groups - 1) * nbuf
        for b in range(nbuf):
            wait_gather(last + b, b)
            fire_write(last + b, b)
        for b in range(nbuf):
            wait_write(last + b, b)

    return gather_kernel


def kernel(x, table):
    B, L = x.shape
    V, D = table.shape
    return _make_gather(V, D, B, L)(x.astype(jnp.int32), table)


# speedup vs baseline: 4.4771x; 4.4771x over previous
import functools

import jax
import jax.numpy as jnp
from jax import lax
from jax.experimental import pallas as pl
from jax.experimental.pallas import tpu as pltpu
from jax.experimental.pallas import tpu_sc as plsc

_NUM_CORES = 2
_NUM_SUBCORES = 16
_NUM_WORKERS = _NUM_CORES * _NUM_SUBCORES


@functools.lru_cache(maxsize=None)
def _make_gather(V, D, B, L, nb=64, nbuf=3):
    N = B * L
    n_per_w = N // _NUM_WORKERS
    n_units = n_per_w // nb
    assert N % _NUM_WORKERS == 0 and n_per_w % nb == 0
    assert B % nb == 0 and nb % 8 == 0 and n_units > nbuf

    mesh = plsc.VectorSubcoreMesh(
        core_axis_name="c", subcore_axis_name="s",
        num_cores=_NUM_CORES, num_subcores=_NUM_SUBCORES)

    @functools.partial(
        pl.kernel,
        out_type=jax.ShapeDtypeStruct((L, B, D), jnp.float32),
        mesh=mesh,
        scratch_types=[
            pltpu.VMEM((n_per_w,), jnp.int32),
            [pltpu.VMEM((nb, D), jnp.float32) for _ in range(nbuf)],
            [pltpu.SemaphoreType.DMA for _ in range(nbuf)],
            [pltpu.SemaphoreType.DMA for _ in range(nbuf)],
        ],
    )
    def gather_kernel(idx_hbm, table_hbm, out_hbm, idx_v, rows, gsem, wsem):
        wid = lax.axis_index("s") * _NUM_CORES + lax.axis_index("c")
        base = wid * n_per_w
        pltpu.sync_copy(idx_hbm.at[pl.ds(base, n_per_w)], idx_v)

        def fire_gather(u, b):
            pltpu.async_copy(
                table_hbm.at[idx_v.at[pl.ds(u * nb, nb)]], rows[b], gsem[b])

        def wait_gather(u, b):
            pltpu.make_async_copy(
                table_hbm.at[idx_v.at[pl.ds(u * nb, nb)]], rows[b],
                gsem[b]).wait()

        def out_slice(u):
            flat = base + u * nb
            return out_hbm.at[flat // B, pl.ds(lax.rem(flat, B), nb)]

        def fire_write(u, b):
            pltpu.async_copy(rows[b], out_slice(u), wsem[b])

        def wait_write(u, b):
            pltpu.make_async_copy(rows[b], out_slice(u), wsem[b]).wait()

        for b in range(nbuf):
            fire_gather(b, b)

        n_main_groups = n_units // nbuf - 1
        def body(i, carry):
            uu = i * nbuf
            for b in range(nbuf):
                u = uu + b
                wait_gather(u, b)
                fire_write(u, b)
                wait_write(u, b)
                fire_gather(u + nbuf, b)
            return carry

        lax.fori_loop(0, n_main_groups, body, 0)

        for u in range(n_main_groups * nbuf, n_units):
            b = u % nbuf
            wait_gather(u, b)
            fire_write(u, b)
            wait_write(u, b)
            if u + nbuf < n_units:
                fire_gather(u + nbuf, b)

    return gather_kernel


def kernel(x, table):
    B, L = x.shape
    V, D = table.shape
    xt = x.T.reshape(B * L).astype(jnp.int32)
    out2 = _make_gather(V, D, B, L)(xt, table)
    return out2.transpose(1, 0, 2)
